# baseline (speedup 1.0000x reference)
import jax
import jax.numpy as jnp
from jax.experimental import pallas as pl
from jax.experimental.pallas import tpu as pltpu

N = 4096
IN_F = 512
OUT_F = 512
ALPHA = 0.2
KC = 3
C_SMALL = -9e-15
D_BIG = -9e15

NPAD = 4608
BM = 512
BF = 512
BFL = 768
BN1 = 1024


def _secondmin_body(hi_ref, hj_ref, sqi_ref, sqj_ref, out_ref, m1_ref, m2_ref):
    j = pl.program_id(1)
    nj = pl.num_programs(1)

    @pl.when(j == 0)
    def _():
        m1_ref[...] = jnp.full_like(m1_ref, jnp.inf)
        m2_ref[...] = jnp.full_like(m2_ref, jnp.inf)

    hj = hj_ref[pl.ds(j * BN1, BN1), :]
    sqj = sqj_ref[:, pl.ds(j * BN1, BN1)]
    dots = jax.lax.dot_general(
        hi_ref[...], hj, (((1,), (1,)), ((), ())),
        preferred_element_type=jnp.float32)
    d = jnp.maximum(sqi_ref[...] + sqj - 2.0 * dots, 0.0)

    t1 = jnp.min(d, axis=1, keepdims=True)
    eq = d == t1
    cnt = jnp.sum(eq.astype(jnp.float32), axis=1, keepdims=True)
    t2_distinct = jnp.min(jnp.where(eq, jnp.inf, d), axis=1, keepdims=True)
    t2 = jnp.where(cnt > 1.0, t1, t2_distinct)

    r1 = m1_ref[...]
    r2 = m2_ref[...]
    m1_ref[...] = jnp.minimum(r1, t1)
    m2_ref[...] = jnp.minimum(jnp.maximum(r1, t1), jnp.minimum(r2, t2))

    @pl.when(j == nj - 1)
    def _():
        out_ref[...] = jnp.sqrt(m2_ref[...])


def _secondmin(h, sq_col, sq_row):
    ni, nj = N // BM, N // BN1
    return pl.pallas_call(
        _secondmin_body,
        grid=(ni, nj),
        in_specs=[
            pl.BlockSpec((BM, IN_F), lambda i, j: (i, 0)),
            pl.BlockSpec((N, IN_F), lambda i, j: (0, 0)),
            pl.BlockSpec((BM, 1), lambda i, j: (i, 0)),
            pl.BlockSpec((1, N), lambda i, j: (0, 0)),
        ],
        out_specs=pl.BlockSpec((BM, 1), lambda i, j: (i, 0)),
        out_shape=jax.ShapeDtypeStruct((N, 1), jnp.float32),
        scratch_shapes=[
            pltpu.VMEM((BM, 1), jnp.float32),
            pltpu.VMEM((BM, 1), jnp.float32),
        ],
    )(h, h, sq_col, sq_row)


def _proj_body(h_ref, w_ref, a1_ref, a2_ref, wh_ref, wh1_ref, wh2_ref):
    i = pl.program_id(0)

    @pl.when(i < N // BF)
    def _():
        wh = jnp.dot(h_ref[...], w_ref[...], preferred_element_type=jnp.float32)
        wh_ref[...] = wh
        wh1_ref[...] = jnp.dot(wh, a1_ref[...],
                               preferred_element_type=jnp.float32)
        wh2_ref[...] = jnp.dot(wh, a2_ref[...],
                               preferred_element_type=jnp.float32)

    @pl.when(i >= N // BF)
    def _():
        wh_ref[...] = jnp.zeros_like(wh_ref)


def _proj(h, W, a1, a2):
    nh = N // BF
    return pl.pallas_call(
        _proj_body,
        grid=(NPAD // BF,),
        in_specs=[
            pl.BlockSpec((BF, IN_F), lambda i: (jnp.minimum(i, nh - 1), 0)),
            pl.BlockSpec((IN_F, OUT_F), lambda i: (0, 0)),
            pl.BlockSpec((OUT_F, 1), lambda i: (0, 0)),
            pl.BlockSpec((OUT_F, 1), lambda i: (0, 0)),
        ],
        out_specs=[
            pl.BlockSpec((BF, OUT_F), lambda i: (i, 0)),
            pl.BlockSpec((BF, 1), lambda i: (jnp.minimum(i, nh - 1), 0)),
            pl.BlockSpec((BF, 1), lambda i: (jnp.minimum(i, nh - 1), 0)),
        ],
        out_shape=[
            jax.ShapeDtypeStruct((NPAD, OUT_F), jnp.float32),
            jax.ShapeDtypeStruct((N, 1), jnp.float32),
            jax.ShapeDtypeStruct((N, 1), jnp.float32),
        ],
    )(h, W, a1, a2)


def _kmeans_centers(data):
    centers = data[:KC]
    for _ in range(10):
        d2 = jnp.sum((data[:, None, :] - centers[None, :, :]) ** 2, axis=2)
        assign = jnp.argmin(d2, axis=1)
        sums = jax.ops.segment_sum(data, assign, num_segments=KC)
        cnt = jax.ops.segment_sum(
            jnp.ones((data.shape[0],), data.dtype), assign, num_segments=KC)
        centers = sums / jnp.maximum(cnt, 1.0)[:, None]
    return centers


def _uv_row_factors(h, dc):
    centers = _kmeans_centers(h)
    d1 = jnp.sqrt(jnp.sum((h[:, None, :] - centers[None, :, :]) ** 2, axis=2))
    idx2 = jnp.arange(N * KC).reshape(N, KC)
    near2 = jnp.take_along_axis(d1, jnp.clip(idx2[:, 1:2], 0, KC - 1), axis=1)
    d1b = jnp.where(d1 != 0.0, dc * near2 / (d1 * d1), d1) - 9e-15
    ones = jnp.ones((N, 1), jnp.float32)
    zer3 = jnp.zeros((N, 3), jnp.float32)
    zer1 = jnp.zeros((N, 1), jnp.float32)
    w4 = jnp.concatenate([ones, d1b, zer3, zer1], axis=1)
    utop = jnp.concatenate([ones, zer3, d1b, zer1], axis=1)
    return w4, utop


def _colsum_body(wh1_ref, wh2t_ref, w4_ref, t2p_ref, t1p_ref,
                 vmain_ref, acct_ref):
    BI = 512
    bn = wh2t_ref.shape[1]
    acc = jnp.zeros((8, bn), jnp.float32)
    wh2t = wh2t_ref[...]
    for i in range(N // BI):
        wh1t = wh1_ref[pl.ds(i * BI, BI), :]
        x = wh1t + wh2t
        e = jnp.where(x >= 0.0, x, ALPHA * x)
        w4t = w4_ref[pl.ds(i * BI, BI), :]
        acc = acc + jax.lax.dot_general(
            w4t, e, (((0,), (0,)), ((), ())),
            preferred_element_type=jnp.float32)
    t2p = t2p_ref[...]
    t2s = jnp.sum(t2p, axis=0, keepdims=True)
    r = jax.lax.broadcasted_iota(jnp.int32, (8, bn), 0)
    sel0 = r == 0
    sel13 = jnp.logical_and(r >= 1, r <= 3)
    vmain_ref[...] = (t2p
                      + jnp.where(sel0, jnp.float32(C_SMALL) * acc, 0.0)
                      + jnp.where(sel13, acc + jnp.float32(D_BIG) * t2s, 0.0))

    @pl.when(pl.program_id(0) == 0)
    def _():
        acct_ref[...] = jax.lax.dot_general(
            w4_ref[...], t1p_ref[...], (((0,), (0,)), ((), ())),
            preferred_element_type=jnp.float32)


def _colsums(wh1, wh2t, w4, t2pad, t1pad):
    BNC = 512
    return pl.pallas_call(
        _colsum_body,
        grid=(N // BNC,),
        in_specs=[
            pl.BlockSpec((N, 1), lambda j: (0, 0)),
            pl.BlockSpec((1, BNC), lambda j: (0, j)),
            pl.BlockSpec((N, 8), lambda j: (0, 0)),
            pl.BlockSpec((8, BNC), lambda j: (0, j)),
            pl.BlockSpec((N, 8), lambda j: (0, 0)),
        ],
        out_specs=[
            pl.BlockSpec((8, BNC), lambda j: (0, j)),
            pl.BlockSpec((8, 8), lambda j: (0, 0)),
        ],
        out_shape=[
            jax.ShapeDtypeStruct((8, N), jnp.float32),
            jax.ShapeDtypeStruct((8, 8), jnp.float32),
        ],
    )(wh1, wh2t, w4, t2pad, t1pad)


def _flash_body(u_ref, v_ref, whp_ref, out_ref, acc_ref, m_ref, l_ref):
    j = pl.program_id(1)
    nj = pl.num_programs(1)

    @pl.when(j == 0)
    def _():
        acc_ref[...] = jnp.zeros_like(acc_ref)
        m_ref[...] = jnp.full_like(m_ref, -jnp.inf)
        l_ref[...] = jnp.zeros_like(l_ref)

    v = v_ref[:, pl.ds(j * BFL, BFL)]
    whp = whp_ref[pl.ds(j * BFL, BFL), :]
    logits = jnp.dot(u_ref[...], v, preferred_element_type=jnp.float32)
    m_prev = m_ref[...]
    m_new = jnp.maximum(m_prev, jnp.max(logits, axis=1, keepdims=True))
    scale = jnp.exp(m_prev - m_new)
    p = jnp.exp(logits - m_new)
    l_ref[...] = l_ref[...] * scale + jnp.sum(p, axis=1, keepdims=True)
    acc_ref[...] = acc_ref[...] * scale + jnp.dot(
        p, whp, preferred_element_type=jnp.float32)
    m_ref[...] = m_new

    @pl.when(j == nj - 1)
    def _():
        hp = acc_ref[...] / l_ref[...]
        out_ref[...] = jnp.where(hp > 0.0, hp, jnp.exp(hp) - 1.0)


def _flash(u, v, whp):
    nb = NPAD // BFL
    return pl.pallas_call(
        _flash_body,
        grid=(nb, nb),
        in_specs=[
            pl.BlockSpec((BFL, 8), lambda i, j: (i, 0)),
            pl.BlockSpec((8, NPAD), lambda i, j: (0, 0)),
            pl.BlockSpec((NPAD, OUT_F), lambda i, j: (0, 0)),
        ],
        out_specs=pl.BlockSpec((BFL, OUT_F), lambda i, j: (i, 0)),
        out_shape=jax.ShapeDtypeStruct((N + KC, OUT_F), jnp.float32),
        scratch_shapes=[
            pltpu.VMEM((BFL, OUT_F), jnp.float32),
            pltpu.VMEM((BFL, 1), jnp.float32),
            pltpu.VMEM((BFL, 1), jnp.float32),
        ],
    )(u, v, whp)


def kernel(h, adj, W, a, tensor1, tensor2):
    del adj
    f32 = jnp.float32
    h = h.astype(f32)

    sq = jnp.sum(h * h, axis=1)
    m2 = _secondmin(h, sq.reshape(N, 1), sq.reshape(1, N))
    dc = jnp.mean(m2)

    w4, utop = _uv_row_factors(h, dc)

    whp, wh1, wh2 = _proj(h, W, a[:OUT_F, :], a[OUT_F:, :])

    t2pad = jnp.concatenate(
        [jnp.zeros((4, N), f32), tensor2, jnp.zeros((1, N), f32)], axis=0)
    t1pad = jnp.concatenate(
        [tensor1, jnp.zeros((N, 8 - KC), f32)], axis=1)
    vmain, acct = _colsums(wh1, wh2.reshape(1, N), w4, t2pad, t1pad)

    npad_tail = NPAD - N - KC
    c = f32(C_SMALL)
    vtail_left = jnp.concatenate(
        [c * acct[0:1, :KC], acct[1:1 + KC, :KC],
         jnp.zeros((8 - 1 - KC, KC), f32)], axis=0)
    vtail_right = jnp.concatenate(
        [jnp.full((1, npad_tail), -1e30, f32),
         jnp.zeros((7, npad_tail), f32)], axis=0)
    v = jnp.concatenate([vmain, vtail_left, vtail_right], axis=1)

    ubot = jnp.zeros((NPAD - N, 8), f32)
    ubot = ubot.at[0:KC, 0].set(1.0)
    for k in range(KC):
        ubot = ubot.at[k, 1 + k].set(1.0)
    u = jnp.concatenate([utop, ubot], axis=0)

    return _flash(u, v, whp)
